# baseline (speedup 1.0000x reference)
import functools

import jax
import jax.numpy as jnp
from jax import lax
from jax.experimental import pallas as pl
from jax.experimental.pallas import tpu as pltpu
from jax.experimental.pallas import tpu_sc as plsc

_B = 16384
_F = 128
_CHUNK = 256


def _sc_predict(pos_i, neg_j, user_table, item_table):
    info = plsc.get_sparse_core_info()
    nc, ns = info.num_cores, info.num_subcores
    nw = nc * ns
    bpw = _B // nw
    nchunk = bpw // _CHUNK
    mesh = plsc.VectorSubcoreMesh(core_axis_name="c", subcore_axis_name="s")

    @functools.partial(
        pl.kernel,
        mesh=mesh,
        compiler_params=pltpu.CompilerParams(needs_layout_passes=False),
        out_type=jax.ShapeDtypeStruct((_B,), jnp.float32),
        scratch_types=[
            pltpu.VMEM((_CHUNK,), jnp.int32),
            pltpu.VMEM((_CHUNK,), jnp.int32),
            pltpu.VMEM((_F,), jnp.float32),
            pltpu.VMEM((_CHUNK, _F), jnp.float32),
            pltpu.VMEM((_CHUNK, _F), jnp.float32),
            pltpu.VMEM((_B // 32,), jnp.float32),
            pltpu.SemaphoreType.DMA,
            pltpu.SemaphoreType.DMA,
        ],
    )
    def body(pos_hbm, neg_hbm, user_hbm, item_hbm, out_hbm,
             idxp_v, idxn_v, user_v, rows_p, rows_n, x_v, sem0, sem1):
        wid = lax.axis_index("s") * nc + lax.axis_index("c")
        base = wid * bpw
        pltpu.sync_copy(user_hbm.at[0], user_v)
        uv = [user_v[pl.ds(16 * k, 16)] for k in range(_F // 16)]
        lane = jnp.arange(16, dtype=jnp.int32)

        for c in range(nchunk):
            off = base + c * _CHUNK
            pltpu.sync_copy(pos_hbm.at[pl.ds(off, _CHUNK)], idxp_v)
            pltpu.sync_copy(neg_hbm.at[pl.ds(off, _CHUNK)], idxn_v)
            cp = pltpu.async_copy(item_hbm.at[idxp_v], rows_p, sem0)
            cn = pltpu.async_copy(item_hbm.at[idxn_v], rows_n, sem1)
            cp.wait()
            cn.wait()

            def group(g, carry):
                idx_row = g * 16 + lane
                acc = jnp.zeros((16,), jnp.float32)
                for f in range(_F):
                    fvec = jnp.full((16,), f, jnp.int32)
                    p = plsc.load_gather(rows_p, [idx_row, fvec])
                    n = plsc.load_gather(rows_n, [idx_row, fvec])
                    acc = acc + (p - n) * uv[f // 16][f % 16]
                x_v[pl.ds(c * _CHUNK + g * 16, 16)] = acc
                return carry

            lax.fori_loop(0, _CHUNK // 16, group, 0)

        pltpu.sync_copy(x_v, out_hbm.at[pl.ds(base, bpw)])

    return body(pos_i, neg_j, user_table, item_table)


def _tc_loss(x):
    def body(x_ref, o_ref):
        z = -x_ref[...]
        sp = jnp.maximum(z, 0.0) + jnp.log1p(jnp.exp(-jnp.abs(z)))
        o_ref[0, 0] = jnp.sum(sp)

    out = pl.pallas_call(
        body,
        out_shape=jax.ShapeDtypeStruct((1, 1), jnp.float32),
        out_specs=pl.BlockSpec(memory_space=pltpu.SMEM),
    )(x.reshape(_B // _F, _F))
    return out[0, 0]


def kernel(n_user, pos_i, neg_j, user_table, item_table):
    x = _sc_predict(pos_i, neg_j, user_table, item_table)
    return _tc_loss(x)


# speedup vs baseline: 1.4436x; 1.4436x over previous
import functools

import jax
import jax.numpy as jnp
from jax import lax
from jax.experimental import pallas as pl
from jax.experimental.pallas import tpu as pltpu
from jax.experimental.pallas import tpu_sc as plsc

_B = 16384
_F = 128
_V = 100000
_ROWS_PER_BLOCK = 4000


def _tc_scores(user_col, item_table):
    nb = _V // _ROWS_PER_BLOCK

    def body(u_ref, rows_ref, s_ref):
        s_ref[...] = jnp.dot(rows_ref[...], u_ref[...],
                             preferred_element_type=jnp.float32)

    out = pl.pallas_call(
        body,
        grid=(nb,),
        in_specs=[
            pl.BlockSpec((_F, 1), lambda i: (0, 0)),
            pl.BlockSpec((_ROWS_PER_BLOCK, _F), lambda i: (i, 0)),
        ],
        out_specs=pl.BlockSpec((_ROWS_PER_BLOCK, 1), lambda i: (i, 0)),
        out_shape=jax.ShapeDtypeStruct((_V, 1), jnp.float32),
    )(user_col, item_table)
    return out.reshape(_V)


def _sc_diff(pos_i, neg_j, scores):
    info = plsc.get_sparse_core_info()
    nc, ns = info.num_cores, info.num_subcores
    nw = nc * ns
    bpw = _B // nw
    mesh = plsc.VectorSubcoreMesh(core_axis_name="c", subcore_axis_name="s")

    @functools.partial(
        pl.kernel,
        mesh=mesh,
        compiler_params=pltpu.CompilerParams(needs_layout_passes=False),
        out_type=jax.ShapeDtypeStruct((_B,), jnp.float32),
        scratch_types=[
            pltpu.VMEM((bpw,), jnp.int32),
            pltpu.VMEM((bpw,), jnp.int32),
            pltpu.VMEM((bpw,), jnp.float32),
            pltpu.VMEM((bpw,), jnp.float32),
            pltpu.SemaphoreType.DMA,
            pltpu.SemaphoreType.DMA,
        ],
    )
    def body(pos_hbm, neg_hbm, s_hbm, out_hbm,
             idxp_v, idxn_v, sp_v, sn_v, sem0, sem1):
        wid = lax.axis_index("s") * nc + lax.axis_index("c")
        base = wid * bpw
        pltpu.sync_copy(pos_hbm.at[pl.ds(base, bpw)], idxp_v)
        pltpu.sync_copy(neg_hbm.at[pl.ds(base, bpw)], idxn_v)
        cp = pltpu.async_copy(s_hbm.at[idxp_v], sp_v, sem0)
        cn = pltpu.async_copy(s_hbm.at[idxn_v], sn_v, sem1)
        cp.wait()
        cn.wait()
        for k in range(bpw // 16):
            sl = pl.ds(16 * k, 16)
            sp_v[sl] = sp_v[sl] - sn_v[sl]
        pltpu.sync_copy(sp_v, out_hbm.at[pl.ds(base, bpw)])

    return body(pos_i, neg_j, scores)


def _tc_loss(x):
    def body(x_ref, o_ref):
        z = -x_ref[...]
        sp = jnp.maximum(z, 0.0) + jnp.log1p(jnp.exp(-jnp.abs(z)))
        o_ref[0, 0] = jnp.sum(sp)

    out = pl.pallas_call(
        body,
        out_shape=jax.ShapeDtypeStruct((1, 1), jnp.float32),
        out_specs=pl.BlockSpec(memory_space=pltpu.SMEM),
    )(x.reshape(_B // _F, _F))
    return out[0, 0]


def kernel(n_user, pos_i, neg_j, user_table, item_table):
    scores = _tc_scores(user_table.reshape(_F, 1), item_table)
    x = _sc_diff(pos_i, neg_j, scores)
    return _tc_loss(x)


# speedup vs baseline: 2.3879x; 1.6541x over previous
import functools

import jax
import jax.numpy as jnp
from jax import lax
from jax.experimental import pallas as pl
from jax.experimental.pallas import tpu as pltpu
from jax.experimental.pallas import tpu_sc as plsc

_B = 16384
_F = 128
_V = 100000
_ROWS_PER_BLOCK = 4096


def _tc_scores(user_row, item_table):
    nb = (_V + _ROWS_PER_BLOCK - 1) // _ROWS_PER_BLOCK
    vpad = nb * _ROWS_PER_BLOCK

    def body(u_ref, rows_ref, s_ref):
        s_ref[...] = lax.dot_general(
            u_ref[...], rows_ref[...], (((1,), (1,)), ((), ())),
            preferred_element_type=jnp.float32)

    out = pl.pallas_call(
        body,
        grid=(nb,),
        in_specs=[
            pl.BlockSpec((1, _F), lambda i: (0, 0)),
            pl.BlockSpec((_ROWS_PER_BLOCK, _F), lambda i: (i, 0)),
        ],
        out_specs=pl.BlockSpec((1, _ROWS_PER_BLOCK), lambda i: (0, i)),
        out_shape=jax.ShapeDtypeStruct((1, vpad), jnp.float32),
    )(user_row, item_table)
    return out.reshape(vpad)


def _sc_diff(pos_i, neg_j, scores):
    info = plsc.get_sparse_core_info()
    nc, ns = info.num_cores, info.num_subcores
    nw = nc * ns
    bpw = _B // nw
    mesh = plsc.VectorSubcoreMesh(core_axis_name="c", subcore_axis_name="s")

    @functools.partial(
        pl.kernel,
        mesh=mesh,
        compiler_params=pltpu.CompilerParams(needs_layout_passes=False),
        out_type=jax.ShapeDtypeStruct((_B,), jnp.float32),
        scratch_types=[
            pltpu.VMEM((bpw,), jnp.int32),
            pltpu.VMEM((bpw,), jnp.int32),
            pltpu.VMEM((bpw,), jnp.float32),
            pltpu.VMEM((bpw,), jnp.float32),
            pltpu.SemaphoreType.DMA,
            pltpu.SemaphoreType.DMA,
        ],
    )
    def body(pos_hbm, neg_hbm, s_hbm, out_hbm,
             idxp_v, idxn_v, sp_v, sn_v, sem0, sem1):
        wid = lax.axis_index("s") * nc + lax.axis_index("c")
        base = wid * bpw
        pltpu.sync_copy(pos_hbm.at[pl.ds(base, bpw)], idxp_v)
        pltpu.sync_copy(neg_hbm.at[pl.ds(base, bpw)], idxn_v)
        cp = pltpu.async_copy(s_hbm.at[idxp_v], sp_v, sem0)
        cn = pltpu.async_copy(s_hbm.at[idxn_v], sn_v, sem1)
        cp.wait()
        cn.wait()
        for k in range(bpw // 16):
            sl = pl.ds(16 * k, 16)
            sp_v[sl] = sp_v[sl] - sn_v[sl]
        pltpu.sync_copy(sp_v, out_hbm.at[pl.ds(base, bpw)])

    return body(pos_i, neg_j, scores)


def _tc_loss(x):
    def body(x_ref, o_ref):
        z = -x_ref[...]
        sp = jnp.maximum(z, 0.0) + jnp.log1p(jnp.exp(-jnp.abs(z)))
        o_ref[0, 0] = jnp.sum(sp)

    out = pl.pallas_call(
        body,
        out_shape=jax.ShapeDtypeStruct((1, 1), jnp.float32),
        out_specs=pl.BlockSpec(memory_space=pltpu.SMEM),
    )(x.reshape(_B // _F, _F))
    return out[0, 0]


def kernel(n_user, pos_i, neg_j, user_table, item_table):
    scores = _tc_scores(user_table, item_table)
    x = _sc_diff(pos_i, neg_j, scores)
    return _tc_loss(x)


# speedup vs baseline: 2.7764x; 1.1627x over previous
import functools

import jax
import jax.numpy as jnp
from jax import lax
from jax.experimental import pallas as pl
from jax.experimental.pallas import tpu as pltpu
from jax.experimental.pallas import tpu_sc as plsc

_B = 16384
_F = 128
_V = 100000
_ROWS_PER_BLOCK = 8192


def _tc_scores(user_row, item_table):
    nb = (_V + _ROWS_PER_BLOCK - 1) // _ROWS_PER_BLOCK
    vpad = nb * _ROWS_PER_BLOCK

    def body(u_ref, rows_ref, s_ref):
        s_ref[...] = lax.dot_general(
            u_ref[...], rows_ref[...], (((1,), (1,)), ((), ())),
            preferred_element_type=jnp.float32)

    out = pl.pallas_call(
        body,
        grid=(nb,),
        in_specs=[
            pl.BlockSpec((1, _F), lambda i: (0, 0)),
            pl.BlockSpec((_ROWS_PER_BLOCK, _F), lambda i: (i, 0)),
        ],
        out_specs=pl.BlockSpec((1, _ROWS_PER_BLOCK), lambda i: (0, i)),
        out_shape=jax.ShapeDtypeStruct((1, vpad), jnp.float32),
    )(user_row, item_table)
    return out.reshape(vpad)


def _sc_diff(pos_i, neg_j, scores):
    info = plsc.get_sparse_core_info()
    nc, ns = info.num_cores, info.num_subcores
    nw = nc * ns
    bpw = _B // nw
    mesh = plsc.VectorSubcoreMesh(core_axis_name="c", subcore_axis_name="s")

    @functools.partial(
        pl.kernel,
        mesh=mesh,
        compiler_params=pltpu.CompilerParams(needs_layout_passes=False),
        out_type=jax.ShapeDtypeStruct((_B,), jnp.float32),
        scratch_types=[
            pltpu.VMEM((bpw,), jnp.int32),
            pltpu.VMEM((bpw,), jnp.int32),
            pltpu.VMEM((bpw,), jnp.float32),
            pltpu.VMEM((bpw,), jnp.float32),
            pltpu.SemaphoreType.DMA,
            pltpu.SemaphoreType.DMA,
        ],
    )
    def body(pos_hbm, neg_hbm, s_hbm, out_hbm,
             idxp_v, idxn_v, sp_v, sn_v, sem0, sem1):
        wid = lax.axis_index("s") * nc + lax.axis_index("c")
        base = wid * bpw
        pltpu.sync_copy(pos_hbm.at[pl.ds(base, bpw)], idxp_v)
        pltpu.sync_copy(neg_hbm.at[pl.ds(base, bpw)], idxn_v)
        cp = pltpu.async_copy(s_hbm.at[idxp_v], sp_v, sem0)
        cn = pltpu.async_copy(s_hbm.at[idxn_v], sn_v, sem1)
        cp.wait()
        cn.wait()
        for k in range(bpw // 16):
            sl = pl.ds(16 * k, 16)
            sp_v[sl] = sp_v[sl] - sn_v[sl]
        pltpu.sync_copy(sp_v, out_hbm.at[pl.ds(base, bpw)])

    return body(pos_i, neg_j, scores)


def _tc_loss(x):
    def body(x_ref, o_ref):
        z = -x_ref[...]
        sp = jnp.maximum(z, 0.0) + jnp.log1p(jnp.exp(-jnp.abs(z)))
        o_ref[0, 0] = jnp.sum(sp)

    out = pl.pallas_call(
        body,
        out_shape=jax.ShapeDtypeStruct((1, 1), jnp.float32),
        out_specs=pl.BlockSpec(memory_space=pltpu.SMEM),
    )(x.reshape(_B // _F, _F))
    return out[0, 0]


def kernel(n_user, pos_i, neg_j, user_table, item_table):
    scores = _tc_scores(user_table, item_table)
    x = _sc_diff(pos_i, neg_j, scores)
    return _tc_loss(x)


# speedup vs baseline: 2.9407x; 1.0592x over previous
import functools

import jax
import jax.numpy as jnp
from jax import lax
from jax.experimental import pallas as pl
from jax.experimental.pallas import tpu as pltpu
from jax.experimental.pallas import tpu_sc as plsc

_B = 16384
_F = 128
_V = 100000
_ROWS_PER_BLOCK = 12800


def _tc_scores(user_row, item_table):
    nb = (_V + _ROWS_PER_BLOCK - 1) // _ROWS_PER_BLOCK
    vpad = nb * _ROWS_PER_BLOCK

    def body(u_ref, rows_ref, s_ref):
        s_ref[...] = lax.dot_general(
            u_ref[...], rows_ref[...], (((1,), (1,)), ((), ())),
            preferred_element_type=jnp.float32)

    out = pl.pallas_call(
        body,
        grid=(nb,),
        in_specs=[
            pl.BlockSpec((1, _F), lambda i: (0, 0)),
            pl.BlockSpec((_ROWS_PER_BLOCK, _F), lambda i: (i, 0)),
        ],
        out_specs=pl.BlockSpec((1, _ROWS_PER_BLOCK), lambda i: (0, i)),
        out_shape=jax.ShapeDtypeStruct((1, vpad), jnp.float32),
    )(user_row, item_table)
    return out.reshape(vpad)


def _sc_diff(pos_i, neg_j, scores):
    info = plsc.get_sparse_core_info()
    nc, ns = info.num_cores, info.num_subcores
    nw = nc * ns
    bpw = _B // nw
    mesh = plsc.VectorSubcoreMesh(core_axis_name="c", subcore_axis_name="s")

    @functools.partial(
        pl.kernel,
        mesh=mesh,
        compiler_params=pltpu.CompilerParams(needs_layout_passes=False),
        out_type=jax.ShapeDtypeStruct((_B,), jnp.float32),
        scratch_types=[
            pltpu.VMEM((bpw,), jnp.int32),
            pltpu.VMEM((bpw,), jnp.int32),
            pltpu.VMEM((bpw,), jnp.float32),
            pltpu.VMEM((bpw,), jnp.float32),
            pltpu.SemaphoreType.DMA,
            pltpu.SemaphoreType.DMA,
        ],
    )
    def body(pos_hbm, neg_hbm, s_hbm, out_hbm,
             idxp_v, idxn_v, sp_v, sn_v, sem0, sem1):
        wid = lax.axis_index("s") * nc + lax.axis_index("c")
        base = wid * bpw
        pltpu.sync_copy(pos_hbm.at[pl.ds(base, bpw)], idxp_v)
        pltpu.sync_copy(neg_hbm.at[pl.ds(base, bpw)], idxn_v)
        cp = pltpu.async_copy(s_hbm.at[idxp_v], sp_v, sem0)
        cn = pltpu.async_copy(s_hbm.at[idxn_v], sn_v, sem1)
        cp.wait()
        cn.wait()
        for k in range(bpw // 16):
            sl = pl.ds(16 * k, 16)
            sp_v[sl] = sp_v[sl] - sn_v[sl]
        pltpu.sync_copy(sp_v, out_hbm.at[pl.ds(base, bpw)])

    return body(pos_i, neg_j, scores)


def _tc_loss(x):
    def body(x_ref, o_ref):
        z = -x_ref[...]
        sp = jnp.maximum(z, 0.0) + jnp.log1p(jnp.exp(-jnp.abs(z)))
        o_ref[0, 0] = jnp.sum(sp)

    out = pl.pallas_call(
        body,
        out_shape=jax.ShapeDtypeStruct((1, 1), jnp.float32),
        out_specs=pl.BlockSpec(memory_space=pltpu.SMEM),
    )(x.reshape(_B // _F, _F))
    return out[0, 0]


def kernel(n_user, pos_i, neg_j, user_table, item_table):
    scores = _tc_scores(user_table, item_table)
    x = _sc_diff(pos_i, neg_j, scores)
    return _tc_loss(x)
